# baseline (speedup 1.0000x reference)
import functools

import jax
import jax.numpy as jnp
from jax import lax
from jax.experimental import pallas as pl
from jax.experimental.pallas import tpu as pltpu
from jax.experimental.pallas import tpu_sc as plsc

N_NODES = 10000
D = 128
N_EDGES = 320000

NC = 2
NS = 16
NW = NC * NS
CHUNK = 128
NCH0 = 160
NCH1 = 0
BASE1 = NS * NCH0
TOT_CH = NS * (NCH0 + NCH1)
E_PAD = TOT_CH * CHUNK
ACC_ROWS = 10240
ROWS_PER_TILE = ACC_ROWS // NS
TRASH_ROW = 10100

_ROW_BLK = 1000


def _mlp1_body(x_ref, w_ref, b_ref, o_ref):
    o_ref[...] = jnp.maximum(
        jnp.dot(x_ref[...], w_ref[...], preferred_element_type=jnp.float32)
        + b_ref[...], 0.0)


def _mlp2_body(p0_ref, p1_ref, x_ref, w_ref, b_ref, o_ref):
    agg = p0_ref[0] + p1_ref[0]
    o_ref[...] = jnp.maximum(
        jnp.dot(agg, w_ref[...], preferred_element_type=jnp.float32)
        + b_ref[...], 0.0) + x_ref[...]


_sc_mesh = plsc.VectorSubcoreMesh(core_axis_name="c", subcore_axis_name="s")


WCH = 8
NWIN0 = NCH0 // WCH
NWIN1 = NCH1 // WCH


@functools.partial(
    pl.kernel,
    mesh=_sc_mesh,
    out_type=jax.ShapeDtypeStruct((NC, ACC_ROWS, D), jnp.float32),
    scratch_types=[
        pltpu.VMEM((2, WCH, CHUNK), jnp.int32),
        pltpu.VMEM((2, WCH, CHUNK), jnp.int32),
        pltpu.VMEM((2, CHUNK, D), jnp.float32),
        pltpu.VMEM_SHARED((ACC_ROWS, D), jnp.float32),
        pltpu.SemaphoreType.DMA,
        pltpu.SemaphoreType.DMA,
        pltpu.SemaphoreType.DMA,
    ],
)
def _sc_aggregate(msg_hbm, src_hbm, dst_hbm, out_hbm, src_w, dst_w, buf_v,
                  acc_sh, semg, sems, semi):
    c = lax.axis_index("c")
    s = lax.axis_index("s")
    start_ch = jnp.where(c == 0, s * NCH0, BASE1 + s * NCH1)
    nwin = jnp.where(c == 0, NWIN0, NWIN1)

    def _zero_row(r, carry):
        for j in range(D // 16):
            buf_v[0, r, pl.ds(j * 16, 16)] = jnp.zeros((16,), jnp.float32)
        return carry
    lax.fori_loop(0, CHUNK, _zero_row, 0)
    for k in range(ROWS_PER_TILE // CHUNK):
        pltpu.sync_copy(buf_v.at[0],
                        acc_sh.at[pl.ds(s * ROWS_PER_TILE + k * CHUNK, CHUNK)])
    plsc.subcore_barrier()

    @pl.when(nwin > 0)
    def _():
        pltpu.sync_copy(src_hbm.at[pl.ds(start_ch, WCH)], src_w.at[0])
        pltpu.sync_copy(dst_hbm.at[pl.ds(start_ch, WCH)], dst_w.at[0])

    def _window(w, carry):
        wb = lax.rem(w, 2)

        @pl.when(w < nwin - 1)
        def _():
            nxt = start_ch + (w + 1) * WCH
            pltpu.async_copy(src_hbm.at[pl.ds(nxt, WCH)],
                             src_w.at[1 - wb], semi)
            pltpu.async_copy(dst_hbm.at[pl.ds(nxt, WCH)],
                             dst_w.at[1 - wb], semi)

        gd = [None, None]
        for k in range(2):
            gd[k] = pltpu.async_copy(msg_hbm.at[src_w.at[wb, k]],
                                     buf_v.at[k], semg)
        for k in range(WCH):
            gd[k % 2].wait()
            sd = pltpu.async_copy(buf_v.at[k % 2],
                                  acc_sh.at[dst_w.at[wb, k]], sems, add=True)
            sd.wait()
            if k + 2 < WCH:
                gd[k % 2] = pltpu.async_copy(msg_hbm.at[src_w.at[wb, k + 2]],
                                             buf_v.at[k % 2], semg)

        @pl.when(w < nwin - 1)
        def _():
            nxt = start_ch + (w + 1) * WCH
            pltpu.make_async_copy(src_hbm.at[pl.ds(nxt, WCH)],
                                  src_w.at[1 - wb], semi).wait()
            pltpu.make_async_copy(dst_hbm.at[pl.ds(nxt, WCH)],
                                  dst_w.at[1 - wb], semi).wait()
        return carry
    lax.fori_loop(0, nwin, _window, 0)
    plsc.subcore_barrier()

    pltpu.sync_copy(acc_sh.at[pl.ds(s * ROWS_PER_TILE, ROWS_PER_TILE)],
                    out_hbm.at[c, pl.ds(s * ROWS_PER_TILE, ROWS_PER_TILE)])


def kernel(x, edge_index, W1, b1, W2, b2):
    src = edge_index[0].astype(jnp.int32)
    dst = edge_index[1].astype(jnp.int32)
    pad = E_PAD - N_EDGES
    src_p = jnp.concatenate([src, jnp.zeros((pad,), jnp.int32)])
    trash = N_NODES + (jnp.arange(pad, dtype=jnp.int32) % (ACC_ROWS - N_NODES))
    dst_p = jnp.concatenate([dst, trash])
    src_p = src_p.reshape(TOT_CH, CHUNK)
    dst_p = dst_p.reshape(TOT_CH, CHUNK)

    messages = pl.pallas_call(
        _mlp1_body,
        grid=(N_NODES // _ROW_BLK,),
        in_specs=[
            pl.BlockSpec((_ROW_BLK, D), lambda i: (i, 0)),
            pl.BlockSpec((D, D), lambda i: (0, 0)),
            pl.BlockSpec((1, D), lambda i: (0, 0)),
        ],
        out_specs=pl.BlockSpec((_ROW_BLK, D), lambda i: (i, 0)),
        out_shape=jax.ShapeDtypeStruct((N_NODES, D), jnp.float32),
    )(x, W1, b1.reshape(1, D))

    partials = _sc_aggregate(messages, src_p, dst_p)

    out = pl.pallas_call(
        _mlp2_body,
        grid=(N_NODES // _ROW_BLK,),
        in_specs=[
            pl.BlockSpec((1, _ROW_BLK, D), lambda i: (0, i, 0)),
            pl.BlockSpec((1, _ROW_BLK, D), lambda i: (1, i, 0)),
            pl.BlockSpec((_ROW_BLK, D), lambda i: (i, 0)),
            pl.BlockSpec((D, D), lambda i: (0, 0)),
            pl.BlockSpec((1, D), lambda i: (0, 0)),
        ],
        out_specs=pl.BlockSpec((_ROW_BLK, D), lambda i: (i, 0)),
        out_shape=jax.ShapeDtypeStruct((N_NODES, D), jnp.float32),
    )(partials, partials, x, W2, b2.reshape(1, D))
    return out


# speedup vs baseline: 1.2465x; 1.2465x over previous
import functools

import jax
import jax.numpy as jnp
from jax import lax
from jax.experimental import pallas as pl
from jax.experimental.pallas import tpu as pltpu
from jax.experimental.pallas import tpu_sc as plsc

N_NODES = 10000
D = 128
N_EDGES = 320000

NC = 2
NS = 16
NW = NC * NS
CHUNK = 128
NCH = 80
E_PAD = NW * NCH * CHUNK
ACC_ROWS = 10240
ROWS_PER_TILE = ACC_ROWS // NS
TRASH_ROW = 10100

_ROW_BLK = 1000


def _mlp1_body(x_ref, w_ref, b_ref, o_ref):
    o_ref[...] = jnp.maximum(
        jnp.dot(x_ref[...], w_ref[...], preferred_element_type=jnp.float32)
        + b_ref[...], 0.0)


def _mlp2_body(p0_ref, p1_ref, x_ref, w_ref, b_ref, o_ref):
    agg = p0_ref[0] + p1_ref[0]
    o_ref[...] = jnp.maximum(
        jnp.dot(agg, w_ref[...], preferred_element_type=jnp.float32)
        + b_ref[...], 0.0) + x_ref[...]


_sc_mesh = plsc.VectorSubcoreMesh(core_axis_name="c", subcore_axis_name="s")


WCH = 8
NWIN = NCH // WCH


@functools.partial(
    pl.kernel,
    mesh=_sc_mesh,
    out_type=jax.ShapeDtypeStruct((NC, ACC_ROWS, D), jnp.float32),
    scratch_types=[
        pltpu.VMEM((2, WCH, CHUNK), jnp.int32),
        pltpu.VMEM((2, WCH, CHUNK), jnp.int32),
        pltpu.VMEM((2, CHUNK, D), jnp.float32),
        pltpu.VMEM_SHARED((ACC_ROWS, D), jnp.float32),
        pltpu.SemaphoreType.DMA,
        pltpu.SemaphoreType.DMA,
        pltpu.SemaphoreType.DMA,
    ],
)
def _sc_aggregate(msg_hbm, src_hbm, dst_hbm, out_hbm, src_w, dst_w, buf_v,
                  acc_sh, semg, sems, semi):
    c = lax.axis_index("c")
    s = lax.axis_index("s")
    wid = s * NC + c

    def _zero_row(r, carry):
        for j in range(D // 16):
            buf_v[0, r, pl.ds(j * 16, 16)] = jnp.zeros((16,), jnp.float32)
        return carry
    lax.fori_loop(0, CHUNK, _zero_row, 0)
    for k in range(ROWS_PER_TILE // CHUNK):
        pltpu.sync_copy(buf_v.at[0],
                        acc_sh.at[pl.ds(s * ROWS_PER_TILE + k * CHUNK, CHUNK)])
    plsc.subcore_barrier()

    pltpu.sync_copy(src_hbm.at[wid, pl.ds(0, WCH)], src_w.at[0])
    pltpu.sync_copy(dst_hbm.at[wid, pl.ds(0, WCH)], dst_w.at[0])

    def _window(w, carry):
        wb = lax.rem(w, 2)

        @pl.when(w < NWIN - 1)
        def _():
            pltpu.async_copy(src_hbm.at[wid, pl.ds((w + 1) * WCH, WCH)],
                             src_w.at[1 - wb], semi)
            pltpu.async_copy(dst_hbm.at[wid, pl.ds((w + 1) * WCH, WCH)],
                             dst_w.at[1 - wb], semi)

        gd = [None, None]
        for k in range(2):
            gd[k] = pltpu.async_copy(msg_hbm.at[src_w.at[wb, k]],
                                     buf_v.at[k], semg)
        for k in range(WCH):
            gd[k % 2].wait()
            sd = pltpu.async_copy(buf_v.at[k % 2],
                                  acc_sh.at[dst_w.at[wb, k]], sems, add=True)
            sd.wait()
            if k + 2 < WCH:
                gd[k % 2] = pltpu.async_copy(msg_hbm.at[src_w.at[wb, k + 2]],
                                             buf_v.at[k % 2], semg)

        @pl.when(w < NWIN - 1)
        def _():
            pltpu.make_async_copy(src_hbm.at[wid, pl.ds((w + 1) * WCH, WCH)],
                                  src_w.at[1 - wb], semi).wait()
            pltpu.make_async_copy(dst_hbm.at[wid, pl.ds((w + 1) * WCH, WCH)],
                                  dst_w.at[1 - wb], semi).wait()
        return carry
    lax.fori_loop(0, NWIN, _window, 0)
    plsc.subcore_barrier()

    pltpu.sync_copy(acc_sh.at[pl.ds(s * ROWS_PER_TILE, ROWS_PER_TILE)],
                    out_hbm.at[c, pl.ds(s * ROWS_PER_TILE, ROWS_PER_TILE)])


def kernel(x, edge_index, W1, b1, W2, b2):
    src = edge_index[0].astype(jnp.int32)
    dst = edge_index[1].astype(jnp.int32)
    pad = E_PAD - N_EDGES
    src_p = jnp.concatenate([src, jnp.zeros((pad,), jnp.int32)])
    trash = N_NODES + (jnp.arange(pad, dtype=jnp.int32) % (ACC_ROWS - N_NODES))
    dst_p = jnp.concatenate([dst, trash])
    src_p = src_p.reshape(NW, NCH, CHUNK)
    dst_p = dst_p.reshape(NW, NCH, CHUNK)

    messages = pl.pallas_call(
        _mlp1_body,
        grid=(N_NODES // _ROW_BLK,),
        in_specs=[
            pl.BlockSpec((_ROW_BLK, D), lambda i: (i, 0)),
            pl.BlockSpec((D, D), lambda i: (0, 0)),
            pl.BlockSpec((1, D), lambda i: (0, 0)),
        ],
        out_specs=pl.BlockSpec((_ROW_BLK, D), lambda i: (i, 0)),
        out_shape=jax.ShapeDtypeStruct((N_NODES, D), jnp.float32),
    )(x, W1, b1.reshape(1, D))

    partials = _sc_aggregate(messages, src_p, dst_p)

    out = pl.pallas_call(
        _mlp2_body,
        grid=(N_NODES // _ROW_BLK,),
        in_specs=[
            pl.BlockSpec((1, _ROW_BLK, D), lambda i: (0, i, 0)),
            pl.BlockSpec((1, _ROW_BLK, D), lambda i: (1, i, 0)),
            pl.BlockSpec((_ROW_BLK, D), lambda i: (i, 0)),
            pl.BlockSpec((D, D), lambda i: (0, 0)),
            pl.BlockSpec((1, D), lambda i: (0, 0)),
        ],
        out_specs=pl.BlockSpec((_ROW_BLK, D), lambda i: (i, 0)),
        out_shape=jax.ShapeDtypeStruct((N_NODES, D), jnp.float32),
    )(partials, partials, x, W2, b2.reshape(1, D))
    return out
